# baseline (speedup 1.0000x reference)
import functools

import jax
import jax.numpy as jnp
from jax import lax
from jax.experimental import pallas as pl
from jax.experimental.pallas import tpu as pltpu, tpu_sc as plsc

N = 10000
E = 640000
D_FEAT = 128
HID = 32
HEADS = 2
P = 200000

NC = 2
NS = 16
NW = NC * NS

N_AL = 10240
EPT = E // NW
ECB = 80
EK = 1
ESB = ECB // EK
ECH = EPT // ECB
ELAST = ECH - 1

PPAD = 204800
PPT = PPAD // NW
PCB = 640
PK = 8
PSB = PCB // PK
PCH = PPT // PCB


def _encode_body(x_ref, wenc_ref, benc_ref, wlin_ref, acat_ref, xh_ref, a4_ref):
    h = jnp.maximum(
        jnp.dot(x_ref[...], wenc_ref[...], preferred_element_type=jnp.float32)
        + benc_ref[0:1, :],
        0.0,
    )
    xh = jnp.dot(h, wlin_ref[...], preferred_element_type=jnp.float32)
    xh_ref[...] = xh
    ps = xh * acat_ref[0:1, :]
    pd = xh * acat_ref[1:2, :]
    a_s0 = jnp.sum(ps[:, :HID], axis=1, keepdims=True)
    a_s1 = jnp.sum(ps[:, HID:], axis=1, keepdims=True)
    a_d0 = jnp.sum(pd[:, :HID], axis=1, keepdims=True)
    a_d1 = jnp.sum(pd[:, HID:], axis=1, keepdims=True)
    t0 = a_s0 + a_d0
    t1 = a_s1 + a_d1
    w0 = jnp.exp(jnp.maximum(t0, 0.2 * t0))
    w1 = jnp.exp(jnp.maximum(t1, 0.2 * t1))
    a4_ref[...] = jnp.concatenate([a_s0, a_s1, a_d0, a_d1, w0, w1, t0, t1], axis=1)


def _encode(x, W_enc, b_enc, W_lin, att_src, att_dst):
    benc2 = jnp.tile(b_enc[None, :], (8, 1))
    acat = jnp.concatenate(
        [att_src.reshape(1, HEADS * HID), att_dst.reshape(1, HEADS * HID)], axis=0
    )
    acat = jnp.tile(acat, (4, 1))
    BLK = 1000
    return pl.pallas_call(
        _encode_body,
        grid=(N // BLK,),
        in_specs=[
            pl.BlockSpec((BLK, D_FEAT), lambda i: (i, 0)),
            pl.BlockSpec((D_FEAT, HID), lambda i: (0, 0)),
            pl.BlockSpec((8, HID), lambda i: (0, 0)),
            pl.BlockSpec((HID, HEADS * HID), lambda i: (0, 0)),
            pl.BlockSpec((8, HEADS * HID), lambda i: (0, 0)),
        ],
        out_specs=[
            pl.BlockSpec((BLK, HEADS * HID), lambda i: (i, 0)),
            pl.BlockSpec((BLK, 8), lambda i: (i, 0)),
        ],
        out_shape=[
            jax.ShapeDtypeStruct((N, HEADS * HID), jnp.float32),
            jax.ShapeDtypeStruct((N, 8), jnp.float32),
        ],
    )(x, W_enc.T, benc2, W_lin.T, acat)


def _edge_kernel(src2d, dst2d, asrc_flat, adst_flat, xh):
    mesh = plsc.VectorSubcoreMesh(
        core_axis_name="c", subcore_axis_name="s", num_cores=NC, num_subcores=NS
    )

    @functools.partial(
        pl.kernel,
        out_type=[
            jax.ShapeDtypeStruct((NC, N_AL, 2 * HID), jnp.float32),
            jax.ShapeDtypeStruct((NC, N_AL, 16), jnp.float32),
        ],
        mesh=mesh,
        compiler_params=pltpu.CompilerParams(
            needs_layout_passes=False, use_tc_tiling_on_sc=False
        ),
        scratch_types=[
            pltpu.VMEM((2 * N,), jnp.float32),
            pltpu.VMEM((2 * N,), jnp.float32),
            pltpu.VMEM((4, EK, ESB), jnp.int32),
            pltpu.VMEM((4, EK, ESB), jnp.int32),
            pltpu.VMEM((ECB, 2 * HID), jnp.float32),
            pltpu.VMEM((ECB, 2 * HID), jnp.float32),
            pltpu.VMEM((ECB, 2 * HID), jnp.float32),
            pltpu.VMEM((ECB, 2 * HID), jnp.float32),
            pltpu.VMEM((ECB, 16), jnp.float32),
            pltpu.VMEM((ECB, 16), jnp.float32),
            pltpu.VMEM_SHARED((N_AL, 2 * HID), jnp.float32),
            pltpu.VMEM_SHARED((N_AL, 16), jnp.float32),
            pltpu.SemaphoreType.DMA,
            pltpu.SemaphoreType.DMA,
            pltpu.SemaphoreType.DMA,
            pltpu.SemaphoreType.DMA,
            pltpu.SemaphoreType.DMA,
            pltpu.SemaphoreType.DMA,
            pltpu.SemaphoreType.DMA,
            pltpu.SemaphoreType.DMA,
        ],
    )
    def body(src_hbm, dst_hbm, asrc_hbm, adst_hbm, xh_hbm, pa_hbm, pd_hbm,
             asrc_v, adst_v, sidx_v, didx_v, rows0, rows1, msg0, msg1, wb0, wb1,
             a_sh, d_sh, is0, is1, is2, is3, gs0, gs1, ss0, ss1):
        rows_b = [rows0, rows1]
        msg_b = [msg0, msg1]
        wb_b = [wb0, wb1]
        isem = [is0, is1, is2, is3]
        gsem = [gs0, gs1]
        ssem = [ss0, ss1]
        c = lax.axis_index("c")
        s = lax.axis_index("s")
        wid = c * NS + s
        lanes = lax.iota(jnp.int32, 16)
        zeros16 = jnp.zeros((16,), jnp.float32)

        pltpu.sync_copy(asrc_hbm, asrc_v)
        pltpu.sync_copy(adst_hbm, adst_v)

        def _zero(r, carry):
            wb0[r, pl.ds(0, 16)] = zeros16
            wb1[r, pl.ds(0, 16)] = zeros16
            for k in range(4):
                rows0[r, pl.ds(k * 16, 16)] = zeros16
            return carry
        lax.fori_loop(0, ECB, _zero, 0)

        z0 = s * (N_AL // NS)
        for q in range(N_AL // NS // ECB):
            pltpu.sync_copy(rows0, a_sh.at[pl.ds(z0 + q * ECB, ECB), :])
            pltpu.sync_copy(wb0, d_sh.at[pl.ds(z0 + q * ECB, ECB), :])
        plsc.subcore_barrier()

        row_base = wid * (EPT // ESB)

        def issue_idx(ch, q):
            r0 = row_base + ch * EK
            pltpu.async_copy(src_hbm.at[pl.ds(r0, EK), :], sidx_v.at[q], isem[q])
            pltpu.async_copy(dst_hbm.at[pl.ds(r0, EK), :], didx_v.at[q], isem[q])

        def wait_idx(q):
            pltpu.make_async_copy(src_hbm.at[pl.ds(0, EK), :], sidx_v.at[q], isem[q]).wait()
            pltpu.make_async_copy(dst_hbm.at[pl.ds(0, EK), :], didx_v.at[q], isem[q]).wait()

        def issue_gather(b, q):
            for k in range(EK):
                pltpu.async_copy(
                    xh_hbm.at[sidx_v.at[q, k]],
                    rows_b[b].at[pl.ds(k * ESB, ESB), :], gsem[b],
                )

        def wait_gather(b, q):
            for k in range(EK):
                pltpu.make_async_copy(
                    xh_hbm.at[sidx_v.at[q, k]],
                    rows_b[b].at[pl.ds(k * ESB, ESB), :], gsem[b],
                ).wait()

        def issue_scatter(b, q):
            for k in range(EK):
                pltpu.async_copy(
                    msg_b[b].at[pl.ds(k * ESB, ESB), :],
                    a_sh.at[didx_v.at[q, k]], ssem[b], add=True,
                )
                pltpu.async_copy(
                    wb_b[b].at[pl.ds(k * ESB, ESB), :],
                    d_sh.at[didx_v.at[q, k]], ssem[b], add=True,
                )

        def wait_scatter(b, q):
            for k in range(EK):
                pltpu.make_async_copy(
                    msg_b[b].at[pl.ds(k * ESB, ESB), :],
                    a_sh.at[didx_v.at[q, k]], ssem[b],
                ).wait()
                pltpu.make_async_copy(
                    wb_b[b].at[pl.ds(k * ESB, ESB), :],
                    d_sh.at[didx_v.at[q, k]], ssem[b],
                ).wait()

        def compute(b, q):
            rows_v = rows_b[b]
            msg_v = msg_b[b]
            wb_v = wb_b[b]

            def _g(g, carry):
                off = g * 16
                sv = sidx_v[q, 0, pl.ds(off, 16)]
                dv = didx_v[q, 0, pl.ds(off, 16)]
                as0 = plsc.load_gather(asrc_v, [2 * sv])
                as1 = plsc.load_gather(asrc_v, [2 * sv + 1])
                ad0 = plsc.load_gather(adst_v, [2 * dv])
                ad1 = plsc.load_gather(adst_v, [2 * dv + 1])
                t0 = as0 + ad0
                t1 = as1 + ad1
                w0 = jnp.exp(jnp.maximum(t0, 0.2 * t0))
                w1 = jnp.exp(jnp.maximum(t1, 0.2 * t1))
                rid = g * 16 + lanes
                for col in range(HID):
                    c16 = jnp.full((16,), col, jnp.int32)
                    val = plsc.load_gather(rows_v, [rid, c16])
                    plsc.store_scatter(msg_v, [rid, c16], val * w0)
                for col in range(HID, 2 * HID):
                    c16 = jnp.full((16,), col, jnp.int32)
                    val = plsc.load_gather(rows_v, [rid, c16])
                    plsc.store_scatter(msg_v, [rid, c16], val * w1)
                plsc.store_scatter(wb_v, [rid, jnp.full((16,), 0, jnp.int32)], w0)
                plsc.store_scatter(wb_v, [rid, jnp.full((16,), 1, jnp.int32)], w1)
                return carry
            lax.fori_loop(0, ECB // 16, _g, 0)

        issue_idx(0, 0)
        wait_idx(0)
        issue_gather(0, 0)
        issue_idx(1, 1)

        def _iter(i, carry):
            for j in range(4):
                ch = i * 4 + j
                b = j % 2

                issue_idx(ch + 2, (j + 2) % 4)

                @pl.when(ch >= 1)
                def _():
                    wait_scatter(1 - b, (j + 3) % 4)

                wait_idx((j + 1) % 4)
                issue_gather(1 - b, (j + 1) % 4)
                wait_gather(b, j)
                compute(b, j)
                issue_scatter(b, j)
            return carry
        lax.fori_loop(0, (ECH - 2) // 4, _iter, 0)

        wait_scatter(1, 3)
        wait_idx(1)
        issue_gather(1, 1)
        wait_gather(0, 0)
        compute(0, 0)
        issue_scatter(0, 0)
        wait_gather(1, 1)
        compute(1, 1)
        issue_scatter(1, 1)
        wait_scatter(0, 0)
        wait_scatter(1, 1)

        plsc.subcore_barrier()
        pltpu.sync_copy(a_sh.at[pl.ds(z0, N_AL // NS), :],
                        pa_hbm.at[c, pl.ds(z0, N_AL // NS), :])
        pltpu.sync_copy(d_sh.at[pl.ds(z0, N_AL // NS), :],
                        pd_hbm.at[c, pl.ds(z0, N_AL // NS), :])

    return body(src2d, dst2d, asrc_flat, adst_flat, xh)


def _combine_body(pa0_ref, pa1_ref, pd0_ref, pd1_ref, xh_ref, a4_ref, bias_ref,
                  w1at_ref, w1bt_ref, b1_ref, u_ref, v_ref):
    A = pa0_ref[...] + pa1_ref[...]
    D = pd0_ref[...] + pd1_ref[...]
    xh = xh_ref[...]
    ws0 = a4_ref[:, 4:5]
    ws1 = a4_ref[:, 5:6]
    num0 = A[:, 0:HID] + ws0 * xh[:, 0:HID]
    num1 = A[:, HID:2 * HID] + ws1 * xh[:, HID:2 * HID]
    den0 = D[:, 0:1] + ws0
    den1 = D[:, 1:2] + ws1
    g = 0.5 * (num0 / den0 + num1 / den1) + bias_ref[0:1, :]
    g = jnp.maximum(g, 0.0)
    ss = jnp.sum(g * g, axis=1, keepdims=True)
    g = g / jnp.maximum(jnp.sqrt(ss), 1e-12)
    u_ref[...] = (
        jnp.dot(g, w1at_ref[...], preferred_element_type=jnp.float32)
        + b1_ref[0:1, :]
    )
    v_ref[...] = jnp.dot(g, w1bt_ref[...], preferred_element_type=jnp.float32)


def _combine(pa0, pa1, pd0, pd1, xh, a4, bias_gat, W1, b1):
    bias2 = jnp.tile(bias_gat[None, :], (8, 1))
    b12 = jnp.tile(b1[None, :], (8, 1))
    w1at = W1[:, :HID].T
    w1bt = W1[:, HID:].T
    BLK = 1000
    return pl.pallas_call(
        _combine_body,
        grid=(N // BLK,),
        in_specs=[
            pl.BlockSpec((BLK, 2 * HID), lambda i: (i, 0)),
            pl.BlockSpec((BLK, 2 * HID), lambda i: (i, 0)),
            pl.BlockSpec((BLK, 16), lambda i: (i, 0)),
            pl.BlockSpec((BLK, 16), lambda i: (i, 0)),
            pl.BlockSpec((BLK, HEADS * HID), lambda i: (i, 0)),
            pl.BlockSpec((BLK, 8), lambda i: (i, 0)),
            pl.BlockSpec((8, HID), lambda i: (0, 0)),
            pl.BlockSpec((HID, HID), lambda i: (0, 0)),
            pl.BlockSpec((HID, HID), lambda i: (0, 0)),
            pl.BlockSpec((8, HID), lambda i: (0, 0)),
        ],
        out_specs=[
            pl.BlockSpec((BLK, HID), lambda i: (i, 0)),
            pl.BlockSpec((BLK, HID), lambda i: (i, 0)),
        ],
        out_shape=[
            jax.ShapeDtypeStruct((N, HID), jnp.float32),
            jax.ShapeDtypeStruct((N, HID), jnp.float32),
        ],
    )(pa0, pa1, pd0, pd1, xh, a4, bias2, w1at, w1bt, b12)


def _pair_kernel(p0_2d, p1_2d, U, V, w2rep, b2rep):
    mesh = plsc.VectorSubcoreMesh(
        core_axis_name="c", subcore_axis_name="s", num_cores=NC, num_subcores=NS
    )

    @functools.partial(
        pl.kernel,
        out_type=jax.ShapeDtypeStruct((PPAD,), jnp.float32),
        mesh=mesh,
        compiler_params=pltpu.CompilerParams(
            needs_layout_passes=False, use_tc_tiling_on_sc=False
        ),
        scratch_types=[
            pltpu.VMEM((PPT // PSB, PSB), jnp.int32),
            pltpu.VMEM((PPT // PSB, PSB), jnp.int32),
            pltpu.VMEM((PCB, HID), jnp.float32),
            pltpu.VMEM((PCB, HID), jnp.float32),
            pltpu.VMEM((PCB, HID), jnp.float32),
            pltpu.VMEM((PCB, HID), jnp.float32),
            pltpu.VMEM((HID, 16), jnp.float32),
            pltpu.VMEM((16,), jnp.float32),
            pltpu.VMEM((PCB,), jnp.float32),
            pltpu.SemaphoreType.DMA,
            pltpu.SemaphoreType.DMA,
        ],
    )
    def body(p0_hbm, p1_hbm, u_hbm, v_hbm, w2_hbm, b2_hbm, scores_hbm,
             i0_v, i1_v, u0, u1, v0, v1, w2_v, b2_v, out_v, gs0, gs1):
        u_b = [u0, u1]
        v_b = [v0, v1]
        gsem = [gs0, gs1]
        c = lax.axis_index("c")
        s = lax.axis_index("s")
        wid = c * NS + s
        lanes = lax.iota(jnp.int32, 16)
        pltpu.sync_copy(w2_hbm, w2_v)
        pltpu.sync_copy(b2_hbm, b2_v)
        irow0 = wid * (PPT // PSB)
        pltpu.sync_copy(p0_hbm.at[pl.ds(irow0, PPT // PSB), :], i0_v)
        pltpu.sync_copy(p1_hbm.at[pl.ds(irow0, PPT // PSB), :], i1_v)

        def issue_gather(ch, b):
            for k in range(PK):
                pltpu.async_copy(
                    u_hbm.at[i0_v.at[ch * PK + k]],
                    u_b[b].at[pl.ds(k * PSB, PSB), :], gsem[b],
                )
                pltpu.async_copy(
                    v_hbm.at[i1_v.at[ch * PK + k]],
                    v_b[b].at[pl.ds(k * PSB, PSB), :], gsem[b],
                )

        def wait_gather(ch, b):
            for k in range(PK):
                pltpu.make_async_copy(
                    u_hbm.at[i0_v.at[ch * PK + k]],
                    u_b[b].at[pl.ds(k * PSB, PSB), :], gsem[b],
                ).wait()
                pltpu.make_async_copy(
                    v_hbm.at[i1_v.at[ch * PK + k]],
                    v_b[b].at[pl.ds(k * PSB, PSB), :], gsem[b],
                ).wait()

        def compute(b):
            def _g(g, carry):
                rid = g * 16 + lanes
                acc = b2_v[...]
                for j in range(HID):
                    j16 = jnp.full((16,), j, jnp.int32)
                    u = plsc.load_gather(u_b[b], [rid, j16])
                    v = plsc.load_gather(v_b[b], [rid, j16])
                    acc = acc + jnp.maximum(u + v, 0.0) * w2_v[j, :]
                out_v[pl.ds(g * 16, 16)] = acc
                return carry
            lax.fori_loop(0, PCB // 16, _g, 0)

        pbase = wid * PPT
        issue_gather(0, 0)
        for ch in range(PCH):
            b = ch % 2
            if ch + 1 < PCH:
                issue_gather(ch + 1, 1 - b)
            wait_gather(ch, b)
            compute(b)
            pltpu.sync_copy(out_v, scores_hbm.at[pl.ds(pbase + ch * PCB, PCB)])

    return body(p0_2d, p1_2d, U, V, w2rep, b2rep)


def kernel(x, edge_index, pair_index, W_enc, b_enc, W_lin, att_src, att_dst,
           bias_gat, W1, b1, W2, b2):
    xh, a4 = _encode(x, W_enc, b_enc, W_lin, att_src, att_dst)
    asrc_flat = a4[:, 0:2].reshape(-1)
    adst_flat = a4[:, 2:4].reshape(-1)

    src2d = edge_index[0].reshape(E // ESB, ESB)
    dst2d = edge_index[1].reshape(E // ESB, ESB)
    parts_a, parts_d = _edge_kernel(src2d, dst2d, asrc_flat, adst_flat, xh)
    U, V = _combine(parts_a[0, :N], parts_a[1, :N], parts_d[0, :N],
                    parts_d[1, :N], xh, a4, bias_gat, W1, b1)

    npad = PPAD - P
    p0 = jnp.concatenate([pair_index[0], jnp.zeros((npad,), jnp.int32)])
    p1 = jnp.concatenate([pair_index[1], jnp.zeros((npad,), jnp.int32)])
    p0_2d = p0.reshape(PPAD // PSB, PSB)
    p1_2d = p1.reshape(PPAD // PSB, PSB)
    w2rep = jnp.tile(W2[0][:, None], (1, 16))
    b2rep = jnp.tile(b2, 16)
    scores = _pair_kernel(p0_2d, p1_2d, U, V, w2rep, b2rep)
    return scores[:P]


# speedup vs baseline: 2.2816x; 2.2816x over previous
import functools

import jax
import jax.numpy as jnp
from jax import lax
from jax.experimental import pallas as pl
from jax.experimental.pallas import tpu as pltpu, tpu_sc as plsc

N = 10000
E = 640000
D_FEAT = 128
HID = 32
HEADS = 2
P = 200000

NC = 2
NS = 16
NW = NC * NS

N_AL = 10240
EPT = E // NW
ECB = 80
EK = 1
ESB = ECB // EK
ECH = EPT // ECB
ELAST = ECH - 1

PPAD = 204800
PPT = PPAD // NW
PCB = 640
PK = 8
PSB = PCB // PK
PCH = PPT // PCB


def _encode_body(x_ref, wenc_ref, benc_ref, wlin_ref, acat_ref, xh_ref, a4_ref):
    h = jnp.maximum(
        jnp.dot(x_ref[...], wenc_ref[...], preferred_element_type=jnp.float32)
        + benc_ref[0:1, :],
        0.0,
    )
    xh = jnp.dot(h, wlin_ref[...], preferred_element_type=jnp.float32)
    xh_ref[...] = xh
    ps = xh * acat_ref[0:1, :]
    pd = xh * acat_ref[1:2, :]
    a_s0 = jnp.sum(ps[:, :HID], axis=1, keepdims=True)
    a_s1 = jnp.sum(ps[:, HID:], axis=1, keepdims=True)
    a_d0 = jnp.sum(pd[:, :HID], axis=1, keepdims=True)
    a_d1 = jnp.sum(pd[:, HID:], axis=1, keepdims=True)
    t0 = a_s0 + a_d0
    t1 = a_s1 + a_d1
    w0 = jnp.exp(jnp.maximum(t0, 0.2 * t0))
    w1 = jnp.exp(jnp.maximum(t1, 0.2 * t1))
    a4_ref[...] = jnp.concatenate([a_s0, a_s1, a_d0, a_d1, w0, w1, t0, t1], axis=1)


def _encode(x, W_enc, b_enc, W_lin, att_src, att_dst):
    benc2 = jnp.tile(b_enc[None, :], (8, 1))
    acat = jnp.concatenate(
        [att_src.reshape(1, HEADS * HID), att_dst.reshape(1, HEADS * HID)], axis=0
    )
    acat = jnp.tile(acat, (4, 1))
    BLK = 1000
    return pl.pallas_call(
        _encode_body,
        grid=(N // BLK,),
        in_specs=[
            pl.BlockSpec((BLK, D_FEAT), lambda i: (i, 0)),
            pl.BlockSpec((D_FEAT, HID), lambda i: (0, 0)),
            pl.BlockSpec((8, HID), lambda i: (0, 0)),
            pl.BlockSpec((HID, HEADS * HID), lambda i: (0, 0)),
            pl.BlockSpec((8, HEADS * HID), lambda i: (0, 0)),
        ],
        out_specs=[
            pl.BlockSpec((BLK, HEADS * HID), lambda i: (i, 0)),
            pl.BlockSpec((BLK, 8), lambda i: (i, 0)),
        ],
        out_shape=[
            jax.ShapeDtypeStruct((N, HEADS * HID), jnp.float32),
            jax.ShapeDtypeStruct((N, 8), jnp.float32),
        ],
    )(x, W_enc.T, benc2, W_lin.T, acat)


def _edge_kernel(src2d, dst2d, asrc_flat, adst_flat, xh):
    mesh = plsc.VectorSubcoreMesh(
        core_axis_name="c", subcore_axis_name="s", num_cores=NC, num_subcores=NS
    )

    @functools.partial(
        pl.kernel,
        out_type=[
            jax.ShapeDtypeStruct((NC, N_AL, 2 * HID), jnp.float32),
            jax.ShapeDtypeStruct((NC, N_AL, 16), jnp.float32),
        ],
        mesh=mesh,
        compiler_params=pltpu.CompilerParams(
            needs_layout_passes=False, use_tc_tiling_on_sc=False
        ),
        scratch_types=[
            pltpu.VMEM((2 * N,), jnp.float32),
            pltpu.VMEM((2 * N,), jnp.float32),
            pltpu.VMEM((4, EK, ESB), jnp.int32),
            pltpu.VMEM((4, EK, ESB), jnp.int32),
            pltpu.VMEM((ECB, 2 * HID), jnp.float32),
            pltpu.VMEM((ECB, 2 * HID), jnp.float32),
            pltpu.VMEM((ECB, 2 * HID), jnp.float32),
            pltpu.VMEM((ECB, 2 * HID), jnp.float32),
            pltpu.VMEM((ECB, 16), jnp.float32),
            pltpu.VMEM((ECB, 16), jnp.float32),
            pltpu.VMEM_SHARED((N_AL, 2 * HID), jnp.float32),
            pltpu.VMEM_SHARED((N_AL, 16), jnp.float32),
            pltpu.SemaphoreType.DMA,
            pltpu.SemaphoreType.DMA,
            pltpu.SemaphoreType.DMA,
            pltpu.SemaphoreType.DMA,
            pltpu.SemaphoreType.DMA,
            pltpu.SemaphoreType.DMA,
            pltpu.SemaphoreType.DMA,
            pltpu.SemaphoreType.DMA,
        ],
    )
    def body(src_hbm, dst_hbm, asrc_hbm, adst_hbm, xh_hbm, pa_hbm, pd_hbm,
             asrc_v, adst_v, sidx_v, didx_v, rows0, rows1, msg0, msg1, wb0, wb1,
             a_sh, d_sh, is0, is1, is2, is3, gs0, gs1, ss0, ss1):
        rows_b = [rows0, rows1]
        msg_b = [msg0, msg1]
        wb_b = [wb0, wb1]
        isem = [is0, is1, is2, is3]
        gsem = [gs0, gs1]
        ssem = [ss0, ss1]
        c = lax.axis_index("c")
        s = lax.axis_index("s")
        wid = c * NS + s
        lanes = lax.iota(jnp.int32, 16)
        zeros16 = jnp.zeros((16,), jnp.float32)

        pltpu.sync_copy(asrc_hbm, asrc_v)
        pltpu.sync_copy(adst_hbm, adst_v)

        def _zero(r, carry):
            wb0[r, pl.ds(0, 16)] = zeros16
            wb1[r, pl.ds(0, 16)] = zeros16
            for k in range(4):
                rows0[r, pl.ds(k * 16, 16)] = zeros16
            return carry
        lax.fori_loop(0, ECB, _zero, 0)

        z0 = s * (N_AL // NS)
        for q in range(N_AL // NS // ECB):
            pltpu.sync_copy(rows0, a_sh.at[pl.ds(z0 + q * ECB, ECB), :])
            pltpu.sync_copy(wb0, d_sh.at[pl.ds(z0 + q * ECB, ECB), :])
        plsc.subcore_barrier()

        row_base = wid * (EPT // ESB)

        def issue_idx(ch, q):
            r0 = row_base + ch * EK
            pltpu.async_copy(src_hbm.at[pl.ds(r0, EK), :], sidx_v.at[q], isem[q])
            pltpu.async_copy(dst_hbm.at[pl.ds(r0, EK), :], didx_v.at[q], isem[q])

        def wait_idx(q):
            pltpu.make_async_copy(src_hbm.at[pl.ds(0, EK), :], sidx_v.at[q], isem[q]).wait()
            pltpu.make_async_copy(dst_hbm.at[pl.ds(0, EK), :], didx_v.at[q], isem[q]).wait()

        def issue_gather(b, q):
            for k in range(EK):
                pltpu.async_copy(
                    xh_hbm.at[sidx_v.at[q, k]],
                    rows_b[b].at[pl.ds(k * ESB, ESB), :], gsem[b],
                )

        def wait_gather(b, q):
            for k in range(EK):
                pltpu.make_async_copy(
                    xh_hbm.at[sidx_v.at[q, k]],
                    rows_b[b].at[pl.ds(k * ESB, ESB), :], gsem[b],
                ).wait()

        def issue_scatter(b, q):
            for k in range(EK):
                pltpu.async_copy(
                    msg_b[b].at[pl.ds(k * ESB, ESB), :],
                    a_sh.at[didx_v.at[q, k]], ssem[b], add=True,
                )
                pltpu.async_copy(
                    wb_b[b].at[pl.ds(k * ESB, ESB), :],
                    d_sh.at[didx_v.at[q, k]], ssem[b], add=True,
                )

        def wait_scatter(b, q):
            for k in range(EK):
                pltpu.make_async_copy(
                    msg_b[b].at[pl.ds(k * ESB, ESB), :],
                    a_sh.at[didx_v.at[q, k]], ssem[b],
                ).wait()
                pltpu.make_async_copy(
                    wb_b[b].at[pl.ds(k * ESB, ESB), :],
                    d_sh.at[didx_v.at[q, k]], ssem[b],
                ).wait()

        def compute(b, q):
            rows_v = rows_b[b]
            msg_v = msg_b[b]
            wb_v = wb_b[b]

            def _g(g, carry):
                off = g * 16
                sv = sidx_v[q, 0, pl.ds(off, 16)]
                dv = didx_v[q, 0, pl.ds(off, 16)]
                as0 = plsc.load_gather(asrc_v, [2 * sv])
                as1 = plsc.load_gather(asrc_v, [2 * sv + 1])
                ad0 = plsc.load_gather(adst_v, [2 * dv])
                ad1 = plsc.load_gather(adst_v, [2 * dv + 1])
                t0 = as0 + ad0
                t1 = as1 + ad1
                w0 = jnp.exp(jnp.maximum(t0, 0.2 * t0))
                w1 = jnp.exp(jnp.maximum(t1, 0.2 * t1))
                rid = g * 16 + lanes
                plsc.store_scatter(wb_v, [rid, jnp.full((16,), 0, jnp.int32)], w0)
                plsc.store_scatter(wb_v, [rid, jnp.full((16,), 1, jnp.int32)], w1)
                for l in range(16):
                    e = off + l
                    l16 = jnp.full((16,), l, jnp.int32)
                    w0s = jnp.take_along_axis(
                        w0, l16, axis=0, mode="promise_in_bounds")
                    w1s = jnp.take_along_axis(
                        w1, l16, axis=0, mode="promise_in_bounds")
                    for k in range(2):
                        msg_v[e, pl.ds(k * 16, 16)] = (
                            rows_v[e, pl.ds(k * 16, 16)] * w0s)
                    for k in range(2, 4):
                        msg_v[e, pl.ds(k * 16, 16)] = (
                            rows_v[e, pl.ds(k * 16, 16)] * w1s)
                return carry
            lax.fori_loop(0, ECB // 16, _g, 0)

        issue_idx(0, 0)
        wait_idx(0)
        issue_gather(0, 0)
        issue_idx(1, 1)

        def _iter(i, carry):
            for j in range(4):
                ch = i * 4 + j
                b = j % 2

                issue_idx(ch + 2, (j + 2) % 4)

                @pl.when(ch >= 1)
                def _():
                    wait_scatter(1 - b, (j + 3) % 4)

                wait_idx((j + 1) % 4)
                issue_gather(1 - b, (j + 1) % 4)
                wait_gather(b, j)
                compute(b, j)
                issue_scatter(b, j)
            return carry
        lax.fori_loop(0, (ECH - 2) // 4, _iter, 0)

        wait_scatter(1, 3)
        wait_idx(1)
        issue_gather(1, 1)
        wait_gather(0, 0)
        compute(0, 0)
        issue_scatter(0, 0)
        wait_gather(1, 1)
        compute(1, 1)
        issue_scatter(1, 1)
        wait_scatter(0, 0)
        wait_scatter(1, 1)

        plsc.subcore_barrier()
        pltpu.sync_copy(a_sh.at[pl.ds(z0, N_AL // NS), :],
                        pa_hbm.at[c, pl.ds(z0, N_AL // NS), :])
        pltpu.sync_copy(d_sh.at[pl.ds(z0, N_AL // NS), :],
                        pd_hbm.at[c, pl.ds(z0, N_AL // NS), :])

    return body(src2d, dst2d, asrc_flat, adst_flat, xh)


def _combine_body(pa0_ref, pa1_ref, pd0_ref, pd1_ref, xh_ref, a4_ref, bias_ref,
                  w1at_ref, w1bt_ref, b1_ref, u_ref, v_ref):
    A = pa0_ref[...] + pa1_ref[...]
    D = pd0_ref[...] + pd1_ref[...]
    xh = xh_ref[...]
    ws0 = a4_ref[:, 4:5]
    ws1 = a4_ref[:, 5:6]
    num0 = A[:, 0:HID] + ws0 * xh[:, 0:HID]
    num1 = A[:, HID:2 * HID] + ws1 * xh[:, HID:2 * HID]
    den0 = D[:, 0:1] + ws0
    den1 = D[:, 1:2] + ws1
    g = 0.5 * (num0 / den0 + num1 / den1) + bias_ref[0:1, :]
    g = jnp.maximum(g, 0.0)
    ss = jnp.sum(g * g, axis=1, keepdims=True)
    g = g / jnp.maximum(jnp.sqrt(ss), 1e-12)
    u_ref[...] = (
        jnp.dot(g, w1at_ref[...], preferred_element_type=jnp.float32)
        + b1_ref[0:1, :]
    )
    v_ref[...] = jnp.dot(g, w1bt_ref[...], preferred_element_type=jnp.float32)


def _combine(pa0, pa1, pd0, pd1, xh, a4, bias_gat, W1, b1):
    bias2 = jnp.tile(bias_gat[None, :], (8, 1))
    b12 = jnp.tile(b1[None, :], (8, 1))
    w1at = W1[:, :HID].T
    w1bt = W1[:, HID:].T
    BLK = 1000
    return pl.pallas_call(
        _combine_body,
        grid=(N // BLK,),
        in_specs=[
            pl.BlockSpec((BLK, 2 * HID), lambda i: (i, 0)),
            pl.BlockSpec((BLK, 2 * HID), lambda i: (i, 0)),
            pl.BlockSpec((BLK, 16), lambda i: (i, 0)),
            pl.BlockSpec((BLK, 16), lambda i: (i, 0)),
            pl.BlockSpec((BLK, HEADS * HID), lambda i: (i, 0)),
            pl.BlockSpec((BLK, 8), lambda i: (i, 0)),
            pl.BlockSpec((8, HID), lambda i: (0, 0)),
            pl.BlockSpec((HID, HID), lambda i: (0, 0)),
            pl.BlockSpec((HID, HID), lambda i: (0, 0)),
            pl.BlockSpec((8, HID), lambda i: (0, 0)),
        ],
        out_specs=[
            pl.BlockSpec((BLK, HID), lambda i: (i, 0)),
            pl.BlockSpec((BLK, HID), lambda i: (i, 0)),
        ],
        out_shape=[
            jax.ShapeDtypeStruct((N, HID), jnp.float32),
            jax.ShapeDtypeStruct((N, HID), jnp.float32),
        ],
    )(pa0, pa1, pd0, pd1, xh, a4, bias2, w1at, w1bt, b12)


def _pair_kernel(p0_2d, p1_2d, U, V, w2rep, b2rep):
    mesh = plsc.VectorSubcoreMesh(
        core_axis_name="c", subcore_axis_name="s", num_cores=NC, num_subcores=NS
    )

    @functools.partial(
        pl.kernel,
        out_type=jax.ShapeDtypeStruct((PPAD,), jnp.float32),
        mesh=mesh,
        compiler_params=pltpu.CompilerParams(
            needs_layout_passes=False, use_tc_tiling_on_sc=False
        ),
        scratch_types=[
            pltpu.VMEM((PPT // PSB, PSB), jnp.int32),
            pltpu.VMEM((PPT // PSB, PSB), jnp.int32),
            pltpu.VMEM((PCB, HID), jnp.float32),
            pltpu.VMEM((PCB, HID), jnp.float32),
            pltpu.VMEM((PCB, HID), jnp.float32),
            pltpu.VMEM((PCB, HID), jnp.float32),
            pltpu.VMEM((HID, 16), jnp.float32),
            pltpu.VMEM((16,), jnp.float32),
            pltpu.VMEM((PCB,), jnp.float32),
            pltpu.SemaphoreType.DMA,
            pltpu.SemaphoreType.DMA,
        ],
    )
    def body(p0_hbm, p1_hbm, u_hbm, v_hbm, w2_hbm, b2_hbm, scores_hbm,
             i0_v, i1_v, u0, u1, v0, v1, w2_v, b2_v, out_v, gs0, gs1):
        u_b = [u0, u1]
        v_b = [v0, v1]
        gsem = [gs0, gs1]
        c = lax.axis_index("c")
        s = lax.axis_index("s")
        wid = c * NS + s
        lanes = lax.iota(jnp.int32, 16)
        pltpu.sync_copy(w2_hbm, w2_v)
        pltpu.sync_copy(b2_hbm, b2_v)
        irow0 = wid * (PPT // PSB)
        pltpu.sync_copy(p0_hbm.at[pl.ds(irow0, PPT // PSB), :], i0_v)
        pltpu.sync_copy(p1_hbm.at[pl.ds(irow0, PPT // PSB), :], i1_v)

        def issue_gather(ch, b):
            for k in range(PK):
                pltpu.async_copy(
                    u_hbm.at[i0_v.at[ch * PK + k]],
                    u_b[b].at[pl.ds(k * PSB, PSB), :], gsem[b],
                )
                pltpu.async_copy(
                    v_hbm.at[i1_v.at[ch * PK + k]],
                    v_b[b].at[pl.ds(k * PSB, PSB), :], gsem[b],
                )

        def wait_gather(ch, b):
            for k in range(PK):
                pltpu.make_async_copy(
                    u_hbm.at[i0_v.at[ch * PK + k]],
                    u_b[b].at[pl.ds(k * PSB, PSB), :], gsem[b],
                ).wait()
                pltpu.make_async_copy(
                    v_hbm.at[i1_v.at[ch * PK + k]],
                    v_b[b].at[pl.ds(k * PSB, PSB), :], gsem[b],
                ).wait()

        def compute(b):
            def _g(g, carry):
                rid = g * 16 + lanes
                acc = b2_v[...]
                for j in range(HID):
                    j16 = jnp.full((16,), j, jnp.int32)
                    u = plsc.load_gather(u_b[b], [rid, j16])
                    v = plsc.load_gather(v_b[b], [rid, j16])
                    acc = acc + jnp.maximum(u + v, 0.0) * w2_v[j, :]
                out_v[pl.ds(g * 16, 16)] = acc
                return carry
            lax.fori_loop(0, PCB // 16, _g, 0)

        pbase = wid * PPT
        issue_gather(0, 0)
        for ch in range(PCH):
            b = ch % 2
            if ch + 1 < PCH:
                issue_gather(ch + 1, 1 - b)
            wait_gather(ch, b)
            compute(b)
            pltpu.sync_copy(out_v, scores_hbm.at[pl.ds(pbase + ch * PCB, PCB)])

    return body(p0_2d, p1_2d, U, V, w2rep, b2rep)


def kernel(x, edge_index, pair_index, W_enc, b_enc, W_lin, att_src, att_dst,
           bias_gat, W1, b1, W2, b2):
    xh, a4 = _encode(x, W_enc, b_enc, W_lin, att_src, att_dst)
    asrc_flat = a4[:, 0:2].reshape(-1)
    adst_flat = a4[:, 2:4].reshape(-1)

    src2d = edge_index[0].reshape(E // ESB, ESB)
    dst2d = edge_index[1].reshape(E // ESB, ESB)
    parts_a, parts_d = _edge_kernel(src2d, dst2d, asrc_flat, adst_flat, xh)
    U, V = _combine(parts_a[0, :N], parts_a[1, :N], parts_d[0, :N],
                    parts_d[1, :N], xh, a4, bias_gat, W1, b1)

    npad = PPAD - P
    p0 = jnp.concatenate([pair_index[0], jnp.zeros((npad,), jnp.int32)])
    p1 = jnp.concatenate([pair_index[1], jnp.zeros((npad,), jnp.int32)])
    p0_2d = p0.reshape(PPAD // PSB, PSB)
    p1_2d = p1.reshape(PPAD // PSB, PSB)
    w2rep = jnp.tile(W2[0][:, None], (1, 16))
    b2rep = jnp.tile(b2, 16)
    scores = _pair_kernel(p0_2d, p1_2d, U, V, w2rep, b2rep)
    return scores[:P]


# speedup vs baseline: 2.6703x; 1.1704x over previous
import functools

import jax
import jax.numpy as jnp
from jax import lax
from jax.experimental import pallas as pl
from jax.experimental.pallas import tpu as pltpu, tpu_sc as plsc

N = 10000
E = 640000
D_FEAT = 128
HID = 32
HEADS = 2
P = 200000

NC = 2
NS = 16
NW = NC * NS

N_AL = 10240
EPT = E // NW
ECB = 80
EK = 1
ESB = ECB // EK
ECH = EPT // ECB
ELAST = ECH - 1

PPAD = 204800
PPT = PPAD // NW
PCB = 640
PK = 8
PSB = PCB // PK
PCH = PPT // PCB


def _encode_body(x_ref, wenc_ref, benc_ref, wlin_ref, acat_ref, xh_ref, a4_ref):
    h = jnp.maximum(
        jnp.dot(x_ref[...], wenc_ref[...], preferred_element_type=jnp.float32)
        + benc_ref[0:1, :],
        0.0,
    )
    xh = jnp.dot(h, wlin_ref[...], preferred_element_type=jnp.float32)
    xh_ref[...] = xh
    ps = xh * acat_ref[0:1, :]
    pd = xh * acat_ref[1:2, :]
    a_s0 = jnp.sum(ps[:, :HID], axis=1, keepdims=True)
    a_s1 = jnp.sum(ps[:, HID:], axis=1, keepdims=True)
    a_d0 = jnp.sum(pd[:, :HID], axis=1, keepdims=True)
    a_d1 = jnp.sum(pd[:, HID:], axis=1, keepdims=True)
    t0 = a_s0 + a_d0
    t1 = a_s1 + a_d1
    w0 = jnp.exp(jnp.maximum(t0, 0.2 * t0))
    w1 = jnp.exp(jnp.maximum(t1, 0.2 * t1))
    a4_ref[...] = jnp.concatenate([a_s0, a_s1, a_d0, a_d1, w0, w1, t0, t1], axis=1)


def _encode(x, W_enc, b_enc, W_lin, att_src, att_dst):
    benc2 = jnp.tile(b_enc[None, :], (8, 1))
    acat = jnp.concatenate(
        [att_src.reshape(1, HEADS * HID), att_dst.reshape(1, HEADS * HID)], axis=0
    )
    acat = jnp.tile(acat, (4, 1))
    BLK = 1000
    return pl.pallas_call(
        _encode_body,
        grid=(N // BLK,),
        in_specs=[
            pl.BlockSpec((BLK, D_FEAT), lambda i: (i, 0)),
            pl.BlockSpec((D_FEAT, HID), lambda i: (0, 0)),
            pl.BlockSpec((8, HID), lambda i: (0, 0)),
            pl.BlockSpec((HID, HEADS * HID), lambda i: (0, 0)),
            pl.BlockSpec((8, HEADS * HID), lambda i: (0, 0)),
        ],
        out_specs=[
            pl.BlockSpec((BLK, HEADS * HID), lambda i: (i, 0)),
            pl.BlockSpec((BLK, 8), lambda i: (i, 0)),
        ],
        out_shape=[
            jax.ShapeDtypeStruct((N, HEADS * HID), jnp.float32),
            jax.ShapeDtypeStruct((N, 8), jnp.float32),
        ],
    )(x, W_enc.T, benc2, W_lin.T, acat)


def _edge_kernel(src2d, dst2d, asrc_flat, adst_flat, xh):
    mesh = plsc.VectorSubcoreMesh(
        core_axis_name="c", subcore_axis_name="s", num_cores=NC, num_subcores=NS
    )

    @functools.partial(
        pl.kernel,
        out_type=[
            jax.ShapeDtypeStruct((NC, N_AL, 2 * HID), jnp.float32),
            jax.ShapeDtypeStruct((NC, N_AL, 16), jnp.float32),
        ],
        mesh=mesh,
        compiler_params=pltpu.CompilerParams(
            needs_layout_passes=False, use_tc_tiling_on_sc=False
        ),
        scratch_types=[
            pltpu.VMEM((2 * N,), jnp.float32),
            pltpu.VMEM((2 * N,), jnp.float32),
            pltpu.VMEM((4, EK, ESB), jnp.int32),
            pltpu.VMEM((4, EK, ESB), jnp.int32),
            pltpu.VMEM((ECB, 2 * HID), jnp.float32),
            pltpu.VMEM((ECB, 2 * HID), jnp.float32),
            pltpu.VMEM((ECB, 2 * HID), jnp.float32),
            pltpu.VMEM((ECB, 2 * HID), jnp.float32),
            pltpu.VMEM((ECB, 16), jnp.float32),
            pltpu.VMEM((ECB, 16), jnp.float32),
            pltpu.VMEM_SHARED((N_AL, 2 * HID), jnp.float32),
            pltpu.VMEM_SHARED((N_AL, 16), jnp.float32),
            pltpu.SemaphoreType.DMA,
            pltpu.SemaphoreType.DMA,
            pltpu.SemaphoreType.DMA,
            pltpu.SemaphoreType.DMA,
            pltpu.SemaphoreType.DMA,
            pltpu.SemaphoreType.DMA,
            pltpu.SemaphoreType.DMA,
            pltpu.SemaphoreType.DMA,
        ],
    )
    def body(src_hbm, dst_hbm, asrc_hbm, adst_hbm, xh_hbm, pa_hbm, pd_hbm,
             asrc_v, adst_v, sidx_v, didx_v, rows0, rows1, msg0, msg1, wb0, wb1,
             a_sh, d_sh, is0, is1, is2, is3, gs0, gs1, ss0, ss1):
        rows_b = [rows0, rows1]
        msg_b = [msg0, msg1]
        wb_b = [wb0, wb1]
        isem = [is0, is1, is2, is3]
        gsem = [gs0, gs1]
        ssem = [ss0, ss1]
        c = lax.axis_index("c")
        s = lax.axis_index("s")
        wid = c * NS + s
        lanes = lax.iota(jnp.int32, 16)
        zeros16 = jnp.zeros((16,), jnp.float32)

        pltpu.sync_copy(asrc_hbm, asrc_v)
        pltpu.sync_copy(adst_hbm, adst_v)

        def _zero(r, carry):
            wb0[r, pl.ds(0, 16)] = zeros16
            wb1[r, pl.ds(0, 16)] = zeros16
            for k in range(4):
                rows0[r, pl.ds(k * 16, 16)] = zeros16
            return carry
        lax.fori_loop(0, ECB, _zero, 0)

        z0 = s * (N_AL // NS)
        for q in range(N_AL // NS // ECB):
            pltpu.sync_copy(rows0, a_sh.at[pl.ds(z0 + q * ECB, ECB), :])
            pltpu.sync_copy(wb0, d_sh.at[pl.ds(z0 + q * ECB, ECB), :])
        plsc.subcore_barrier()

        row_base = wid * (EPT // ESB)

        def issue_idx(ch, q):
            r0 = row_base + ch * EK
            pltpu.async_copy(src_hbm.at[pl.ds(r0, EK), :], sidx_v.at[q], isem[q])
            pltpu.async_copy(dst_hbm.at[pl.ds(r0, EK), :], didx_v.at[q], isem[q])

        def wait_idx(q):
            pltpu.make_async_copy(src_hbm.at[pl.ds(0, EK), :], sidx_v.at[q], isem[q]).wait()
            pltpu.make_async_copy(dst_hbm.at[pl.ds(0, EK), :], didx_v.at[q], isem[q]).wait()

        def issue_gather(b, q):
            for k in range(EK):
                pltpu.async_copy(
                    xh_hbm.at[sidx_v.at[q, k]],
                    rows_b[b].at[pl.ds(k * ESB, ESB), :], gsem[b],
                )

        def wait_gather(b, q):
            for k in range(EK):
                pltpu.make_async_copy(
                    xh_hbm.at[sidx_v.at[q, k]],
                    rows_b[b].at[pl.ds(k * ESB, ESB), :], gsem[b],
                ).wait()

        def issue_scatter(b, q):
            for k in range(EK):
                pltpu.async_copy(
                    msg_b[b].at[pl.ds(k * ESB, ESB), :],
                    a_sh.at[didx_v.at[q, k]], ssem[b], add=True,
                )
                pltpu.async_copy(
                    wb_b[b].at[pl.ds(k * ESB, ESB), :],
                    d_sh.at[didx_v.at[q, k]], ssem[b], add=True,
                )

        def wait_scatter(b, q):
            for k in range(EK):
                pltpu.make_async_copy(
                    msg_b[b].at[pl.ds(k * ESB, ESB), :],
                    a_sh.at[didx_v.at[q, k]], ssem[b],
                ).wait()
                pltpu.make_async_copy(
                    wb_b[b].at[pl.ds(k * ESB, ESB), :],
                    d_sh.at[didx_v.at[q, k]], ssem[b],
                ).wait()

        def compute(b, q):
            rows_v = rows_b[b]
            msg_v = msg_b[b]
            wb_v = wb_b[b]

            def _g(g, carry):
                off = g * 16
                sv = sidx_v[q, 0, pl.ds(off, 16)]
                dv = didx_v[q, 0, pl.ds(off, 16)]
                as0 = plsc.load_gather(asrc_v, [2 * sv])
                as1 = plsc.load_gather(asrc_v, [2 * sv + 1])
                ad0 = plsc.load_gather(adst_v, [2 * dv])
                ad1 = plsc.load_gather(adst_v, [2 * dv + 1])
                t0 = as0 + ad0
                t1 = as1 + ad1
                w0 = jnp.exp(jnp.maximum(t0, 0.2 * t0))
                w1 = jnp.exp(jnp.maximum(t1, 0.2 * t1))
                rid = g * 16 + lanes
                plsc.store_scatter(wb_v, [rid, jnp.full((16,), 0, jnp.int32)], w0)
                plsc.store_scatter(wb_v, [rid, jnp.full((16,), 1, jnp.int32)], w1)
                for l in range(16):
                    e = off + l
                    l16 = jnp.full((16,), l, jnp.int32)
                    w0s = jnp.take_along_axis(
                        w0, l16, axis=0, mode="promise_in_bounds")
                    w1s = jnp.take_along_axis(
                        w1, l16, axis=0, mode="promise_in_bounds")
                    for k in range(2):
                        msg_v[e, pl.ds(k * 16, 16)] = (
                            rows_v[e, pl.ds(k * 16, 16)] * w0s)
                    for k in range(2, 4):
                        msg_v[e, pl.ds(k * 16, 16)] = (
                            rows_v[e, pl.ds(k * 16, 16)] * w1s)
                return carry
            lax.fori_loop(0, ECB // 16, _g, 0)

        issue_idx(0, 0)
        wait_idx(0)
        issue_gather(0, 0)
        issue_idx(1, 1)

        def _iter(i, carry):
            for j in range(4):
                ch = i * 4 + j
                b = j % 2

                issue_idx(ch + 2, (j + 2) % 4)

                @pl.when(ch >= 1)
                def _():
                    wait_scatter(1 - b, (j + 3) % 4)

                wait_idx((j + 1) % 4)
                issue_gather(1 - b, (j + 1) % 4)
                wait_gather(b, j)
                compute(b, j)
                issue_scatter(b, j)
            return carry
        lax.fori_loop(0, (ECH - 2) // 4, _iter, 0)

        wait_scatter(1, 3)
        wait_idx(1)
        issue_gather(1, 1)
        wait_gather(0, 0)
        compute(0, 0)
        issue_scatter(0, 0)
        wait_gather(1, 1)
        compute(1, 1)
        issue_scatter(1, 1)
        wait_scatter(0, 0)
        wait_scatter(1, 1)

        plsc.subcore_barrier()
        pltpu.sync_copy(a_sh.at[pl.ds(z0, N_AL // NS), :],
                        pa_hbm.at[c, pl.ds(z0, N_AL // NS), :])
        pltpu.sync_copy(d_sh.at[pl.ds(z0, N_AL // NS), :],
                        pd_hbm.at[c, pl.ds(z0, N_AL // NS), :])

    return body(src2d, dst2d, asrc_flat, adst_flat, xh)


def _combine_body(pa0_ref, pa1_ref, pd0_ref, pd1_ref, xh_ref, a4_ref, bias_ref,
                  w1at_ref, w1bt_ref, b1_ref, u_ref, v_ref):
    A = pa0_ref[...] + pa1_ref[...]
    D = pd0_ref[...] + pd1_ref[...]
    xh = xh_ref[...]
    ws0 = a4_ref[:, 4:5]
    ws1 = a4_ref[:, 5:6]
    num0 = A[:, 0:HID] + ws0 * xh[:, 0:HID]
    num1 = A[:, HID:2 * HID] + ws1 * xh[:, HID:2 * HID]
    den0 = D[:, 0:1] + ws0
    den1 = D[:, 1:2] + ws1
    g = 0.5 * (num0 / den0 + num1 / den1) + bias_ref[0:1, :]
    g = jnp.maximum(g, 0.0)
    ss = jnp.sum(g * g, axis=1, keepdims=True)
    g = g / jnp.maximum(jnp.sqrt(ss), 1e-12)
    u_ref[...] = (
        jnp.dot(g, w1at_ref[...], preferred_element_type=jnp.float32)
        + b1_ref[0:1, :]
    )
    v_ref[...] = jnp.dot(g, w1bt_ref[...], preferred_element_type=jnp.float32)


def _combine(pa0, pa1, pd0, pd1, xh, a4, bias_gat, W1, b1):
    bias2 = jnp.tile(bias_gat[None, :], (8, 1))
    b12 = jnp.tile(b1[None, :], (8, 1))
    w1at = W1[:, :HID].T
    w1bt = W1[:, HID:].T
    BLK = 1000
    return pl.pallas_call(
        _combine_body,
        grid=(N // BLK,),
        in_specs=[
            pl.BlockSpec((BLK, 2 * HID), lambda i: (i, 0)),
            pl.BlockSpec((BLK, 2 * HID), lambda i: (i, 0)),
            pl.BlockSpec((BLK, 16), lambda i: (i, 0)),
            pl.BlockSpec((BLK, 16), lambda i: (i, 0)),
            pl.BlockSpec((BLK, HEADS * HID), lambda i: (i, 0)),
            pl.BlockSpec((BLK, 8), lambda i: (i, 0)),
            pl.BlockSpec((8, HID), lambda i: (0, 0)),
            pl.BlockSpec((HID, HID), lambda i: (0, 0)),
            pl.BlockSpec((HID, HID), lambda i: (0, 0)),
            pl.BlockSpec((8, HID), lambda i: (0, 0)),
        ],
        out_specs=[
            pl.BlockSpec((BLK, HID), lambda i: (i, 0)),
            pl.BlockSpec((BLK, HID), lambda i: (i, 0)),
        ],
        out_shape=[
            jax.ShapeDtypeStruct((N, HID), jnp.float32),
            jax.ShapeDtypeStruct((N, HID), jnp.float32),
        ],
    )(pa0, pa1, pd0, pd1, xh, a4, bias2, w1at, w1bt, b12)


def _pair_kernel(p0_2d, p1_2d, U, V, w2rep, b2rep):
    mesh = plsc.VectorSubcoreMesh(
        core_axis_name="c", subcore_axis_name="s", num_cores=NC, num_subcores=NS
    )

    @functools.partial(
        pl.kernel,
        out_type=jax.ShapeDtypeStruct((PPAD,), jnp.float32),
        mesh=mesh,
        compiler_params=pltpu.CompilerParams(
            needs_layout_passes=False, use_tc_tiling_on_sc=False
        ),
        scratch_types=[
            pltpu.VMEM((PPT // PSB, PSB), jnp.int32),
            pltpu.VMEM((PPT // PSB, PSB), jnp.int32),
            pltpu.VMEM((PCB, HID), jnp.float32),
            pltpu.VMEM((PCB, HID), jnp.float32),
            pltpu.VMEM((PCB, HID), jnp.float32),
            pltpu.VMEM((PCB, HID), jnp.float32),
            pltpu.VMEM((HID,), jnp.float32),
            pltpu.VMEM((16,), jnp.float32),
            pltpu.VMEM((PCB,), jnp.float32),
            pltpu.SemaphoreType.DMA,
            pltpu.SemaphoreType.DMA,
        ],
    )
    def body(p0_hbm, p1_hbm, u_hbm, v_hbm, w2_hbm, b2_hbm, scores_hbm,
             i0_v, i1_v, u0, u1, v0, v1, w2_v, b2_v, out_v, gs0, gs1):
        u_b = [u0, u1]
        v_b = [v0, v1]
        gsem = [gs0, gs1]
        c = lax.axis_index("c")
        s = lax.axis_index("s")
        wid = c * NS + s
        lanes = lax.iota(jnp.int32, 16)
        pltpu.sync_copy(w2_hbm, w2_v)
        pltpu.sync_copy(b2_hbm, b2_v)
        irow0 = wid * (PPT // PSB)
        pltpu.sync_copy(p0_hbm.at[pl.ds(irow0, PPT // PSB), :], i0_v)
        pltpu.sync_copy(p1_hbm.at[pl.ds(irow0, PPT // PSB), :], i1_v)

        def issue_gather(ch, b):
            for k in range(PK):
                pltpu.async_copy(
                    u_hbm.at[i0_v.at[ch * PK + k]],
                    u_b[b].at[pl.ds(k * PSB, PSB), :], gsem[b],
                )
                pltpu.async_copy(
                    v_hbm.at[i1_v.at[ch * PK + k]],
                    v_b[b].at[pl.ds(k * PSB, PSB), :], gsem[b],
                )

        def wait_gather(ch, b):
            for k in range(PK):
                pltpu.make_async_copy(
                    u_hbm.at[i0_v.at[ch * PK + k]],
                    u_b[b].at[pl.ds(k * PSB, PSB), :], gsem[b],
                ).wait()
                pltpu.make_async_copy(
                    v_hbm.at[i1_v.at[ch * PK + k]],
                    v_b[b].at[pl.ds(k * PSB, PSB), :], gsem[b],
                ).wait()

        xshifts = [jnp.bitwise_xor(lanes, m) for m in (8, 4, 2, 1)]

        def compute(b):
            w2a = w2_v[pl.ds(0, 16)]
            w2b = w2_v[pl.ds(16, 16)]
            b2s = b2_v[...]

            def _g(g, carry):
                off = g * 16
                acc = jnp.zeros((16,), jnp.float32)
                for l in range(16):
                    q = off + l
                    s0 = jnp.maximum(
                        u_b[b][q, pl.ds(0, 16)] + v_b[b][q, pl.ds(0, 16)], 0.0)
                    s1 = jnp.maximum(
                        u_b[b][q, pl.ds(16, 16)] + v_b[b][q, pl.ds(16, 16)], 0.0)
                    t = s0 * w2a + s1 * w2b
                    for xc in xshifts:
                        t = t + jnp.take_along_axis(
                            t, xc, axis=0, mode="promise_in_bounds")
                    acc = jnp.where(lanes == l, t, acc)
                out_v[pl.ds(off, 16)] = acc + b2s
                return carry
            lax.fori_loop(0, PCB // 16, _g, 0)

        pbase = wid * PPT
        issue_gather(0, 0)
        for ch in range(PCH):
            b = ch % 2
            if ch + 1 < PCH:
                issue_gather(ch + 1, 1 - b)
            wait_gather(ch, b)
            compute(b)
            pltpu.sync_copy(out_v, scores_hbm.at[pl.ds(pbase + ch * PCB, PCB)])

    return body(p0_2d, p1_2d, U, V, w2rep, b2rep)


def kernel(x, edge_index, pair_index, W_enc, b_enc, W_lin, att_src, att_dst,
           bias_gat, W1, b1, W2, b2):
    xh, a4 = _encode(x, W_enc, b_enc, W_lin, att_src, att_dst)
    asrc_flat = a4[:, 0:2].reshape(-1)
    adst_flat = a4[:, 2:4].reshape(-1)

    src2d = edge_index[0].reshape(E // ESB, ESB)
    dst2d = edge_index[1].reshape(E // ESB, ESB)
    parts_a, parts_d = _edge_kernel(src2d, dst2d, asrc_flat, adst_flat, xh)
    U, V = _combine(parts_a[0, :N], parts_a[1, :N], parts_d[0, :N],
                    parts_d[1, :N], xh, a4, bias_gat, W1, b1)

    npad = PPAD - P
    p0 = jnp.concatenate([pair_index[0], jnp.zeros((npad,), jnp.int32)])
    p1 = jnp.concatenate([pair_index[1], jnp.zeros((npad,), jnp.int32)])
    p0_2d = p0.reshape(PPAD // PSB, PSB)
    p1_2d = p1.reshape(PPAD // PSB, PSB)
    b2rep = jnp.tile(b2, 16)
    scores = _pair_kernel(p0_2d, p1_2d, U, V, W2[0], b2rep)
    return scores[:P]
